# Initial kernel scaffold; baseline (speedup 1.0000x reference)
import jax
import jax.numpy as jnp
from jax.experimental import pallas as pl


def kernel(inputs, rpbt):
    raise NotImplementedError("write your pallas kernel here")


# speedup vs baseline: 1.6838x; 1.6838x over previous
import functools

import numpy as np
import jax
import jax.numpy as jnp
from jax import lax
from jax.experimental import pallas as pl
from jax.experimental.pallas import tpu as pltpu
from jax.experimental.pallas import tpu_sc as plsc

_NH = 16
_NRD = (2 * 8 - 1) ** 3 + 3
_VP1 = 8 * 8 * 8 + 1
_N = _VP1 * _VP1
_NW = 32
_PER = 8192
_CHUNK = 2048
_NCHUNK = _PER // _CHUNK
_XTRA = (_N - _NW * _PER) // 128
_XBASE = _NW * _PER
_LAST = _XBASE + _XTRA * 128
_IDX_PAD_LEN = _LAST + 16


def _rel_pos_index_flat() -> np.ndarray:
    Wh = Ww = Wd = 8
    coords = np.stack(np.meshgrid(np.arange(Wh), np.arange(Ww), np.arange(Wd),
                                  indexing="ij"))
    cf = coords.reshape(3, -1)
    rel = (cf[:, :, None] - cf[:, None, :]).transpose(1, 2, 0).astype(np.int64)
    rel[:, :, 0] += Wh - 1
    rel[:, :, 1] += Ww - 1
    rel[:, :, 2] += Wd - 1
    rel[:, :, 0] *= (2 * Ww - 1) * (2 * Wd - 1)
    rel[:, :, 1] *= 2 * Wd - 1
    idx = rel.sum(-1)
    V = Wh * Ww * Wd
    nrd = (2 * Wh - 1) * (2 * Ww - 1) * (2 * Wd - 1) + 3
    full = np.zeros((V + 1, V + 1), np.int32)
    full[1:, 1:] = idx
    full[0, :] = nrd - 3
    full[:, 0] = nrd - 2
    full[0, 0] = nrd - 1
    flat = np.zeros((_IDX_PAD_LEN,), np.int32)
    flat[:_N] = full.reshape(-1)
    return flat


_IDX_FLAT = _rel_pos_index_flat()

_mesh = plsc.VectorSubcoreMesh(core_axis_name="c", subcore_axis_name="s")


@functools.partial(
    pl.kernel,
    mesh=_mesh,
    out_type=jax.ShapeDtypeStruct((_NH, _N), jnp.float32),
    compiler_params=pltpu.CompilerParams(
        needs_layout_passes=False, use_tc_tiling_on_sc=False),
    scratch_types=[
        pltpu.VMEM((_NH * _NRD,), jnp.float32),
        pltpu.VMEM((_PER,), jnp.int32),
        pltpu.VMEM((_NH, _CHUNK), jnp.float32),
        pltpu.VMEM((128,), jnp.int32),
        pltpu.VMEM((_NH, 128), jnp.float32),
        pltpu.VMEM((16,), jnp.int32),
        pltpu.VMEM((_NH, 16), jnp.float32),
    ],
)
def _sc_gather(tab_hbm, idx_hbm, out_hbm,
               tab_v, idx_v, buf, xidx, xbuf, lidx, lbuf):
    w = lax.axis_index("s") * 2 + lax.axis_index("c")
    base = w * _PER
    pltpu.sync_copy(tab_hbm, tab_v)
    pltpu.sync_copy(idx_hbm.at[pl.ds(base, _PER)], idx_v)

    def gather_vec(vec):
        return [plsc.load_gather(tab_v, [vec + h * _NRD]) for h in range(_NH)]

    for c in range(_NCHUNK):
        def body(i, carry, c=c):
            vec = idx_v[pl.ds((c * (_CHUNK // 16) + i) * 16, 16)]
            vals = gather_vec(vec)
            for h in range(_NH):
                buf[h, pl.ds(i * 16, 16)] = vals[h]
            return carry
        lax.fori_loop(0, _CHUNK // 16, body, 0)
        pltpu.sync_copy(buf, out_hbm.at[:, pl.ds(base + c * _CHUNK, _CHUNK)])

    @pl.when(w < _XTRA)
    def _extra():
        xb = _XBASE + w * 128
        pltpu.sync_copy(idx_hbm.at[pl.ds(xb, 128)], xidx)

        def body(i, carry):
            vec = xidx[pl.ds(i * 16, 16)]
            vals = gather_vec(vec)
            for h in range(_NH):
                xbuf[h, pl.ds(i * 16, 16)] = vals[h]
            return carry
        lax.fori_loop(0, 8, body, 0)
        pltpu.sync_copy(xbuf, out_hbm.at[:, pl.ds(xb, 128)])

    @pl.when(w == _XTRA)
    def _leftover():
        pltpu.sync_copy(idx_hbm.at[pl.ds(_LAST, 16)], lidx)
        vals = gather_vec(lidx[...])
        for h in range(_NH):
            lbuf[h, :] = vals[h]
        pltpu.sync_copy(lbuf.at[:, pl.ds(0, 1)],
                        out_hbm.at[:, pl.ds(_LAST, 1)])


def kernel(inputs, rpbt):
    del inputs
    tab = jnp.transpose(rpbt).reshape(-1)
    idx = jnp.asarray(_IDX_FLAT)
    out = _sc_gather(tab, idx)
    return out.reshape(_NH, _VP1, _VP1)


# speedup vs baseline: 4.3969x; 2.6113x over previous
import functools

import numpy as np
import jax
import jax.numpy as jnp
from jax import lax
from jax.experimental import pallas as pl
from jax.experimental.pallas import tpu as pltpu
from jax.experimental.pallas import tpu_sc as plsc

_NH = 16
_NRD = (2 * 8 - 1) ** 3 + 3
_VP1 = 8 * 8 * 8 + 1
_RP = 528
_NVROW = _RP // 16
_BR = 8
_NBLK = (_VP1 - 1) // _BR
_NW = 32
_BLK_IDX = _BR * _RP


def _rel_pos_index_padded() -> np.ndarray:
    Wh = Ww = Wd = 8
    coords = np.stack(np.meshgrid(np.arange(Wh), np.arange(Ww), np.arange(Wd),
                                  indexing="ij"))
    cf = coords.reshape(3, -1)
    rel = (cf[:, :, None] - cf[:, None, :]).transpose(1, 2, 0).astype(np.int64)
    rel[:, :, 0] += Wh - 1
    rel[:, :, 1] += Ww - 1
    rel[:, :, 2] += Wd - 1
    rel[:, :, 0] *= (2 * Ww - 1) * (2 * Wd - 1)
    rel[:, :, 1] *= 2 * Wd - 1
    idx = rel.sum(-1)
    V = Wh * Ww * Wd
    nrd = (2 * Wh - 1) * (2 * Ww - 1) * (2 * Wd - 1) + 3
    full = np.zeros((V + 1, V + 1), np.int32)
    full[1:, 1:] = idx
    full[0, :] = nrd - 3
    full[:, 0] = nrd - 2
    full[0, 0] = nrd - 1
    padded = np.zeros((_VP1, _RP), np.int32)
    padded[:, :_VP1] = full
    return padded.reshape(-1)


_IDX_FLAT = _rel_pos_index_padded()

_mesh = plsc.VectorSubcoreMesh(core_axis_name="c", subcore_axis_name="s")


@functools.partial(
    pl.kernel,
    mesh=_mesh,
    out_type=jax.ShapeDtypeStruct((_NH, _VP1, _VP1), jnp.float32),
    compiler_params=pltpu.CompilerParams(
        needs_layout_passes=False, use_tc_tiling_on_sc=False),
    scratch_types=[
        pltpu.VMEM((_NH * _NRD,), jnp.float32),
        pltpu.VMEM((_BLK_IDX,), jnp.int32),
        pltpu.VMEM((_NH, _BR, _VP1), jnp.float32),
    ],
)
def _sc_gather(tab_hbm, idx_hbm, out_hbm, tab_v, idx_v, buf):
    w = lax.axis_index("s") * 2 + lax.axis_index("c")
    pltpu.sync_copy(tab_hbm, tab_v)

    def gather_vec(vec):
        return [plsc.load_gather(tab_v, [vec + h * _NRD]) for h in range(_NH)]

    lane = lax.broadcasted_iota(jnp.int32, (16,), 0)
    lane0 = lane < 1
    col_last = jnp.full((16,), _VP1 - 1, jnp.int32)

    def do_block(nrows):
        def body(v, carry):
            for r in range(nrows):
                vec = idx_v[pl.ds(r * _RP + v * 16, 16)]
                vals = gather_vec(vec)
                for h in range(_NH):
                    buf[h, r, pl.ds(v * 16, 16)] = vals[h]
            return carry
        lax.fori_loop(0, (_VP1 - 1) // 16, body, 0)
        for r in range(nrows):
            vec = idx_v[pl.ds(r * _RP + _VP1 - 1, 16)]
            vals = gather_vec(vec)
            row = jnp.full((16,), r, jnp.int32)
            for h in range(_NH):
                plsc.store_scatter(buf, [jnp.full((16,), h, jnp.int32), row,
                                         col_last], vals[h], mask=lane0)

    for b in range(2):
        blk = w * 2 + b
        pltpu.sync_copy(idx_hbm.at[pl.ds(blk * _BLK_IDX, _BLK_IDX)], idx_v)
        do_block(_BR)
        pltpu.sync_copy(buf, out_hbm.at[:, pl.ds(blk * _BR, _BR), :])

    @pl.when(w == _NW - 1)
    def _last_row():
        pltpu.sync_copy(idx_hbm.at[pl.ds(_NBLK * _BLK_IDX, _RP)],
                        idx_v.at[pl.ds(0, _RP)])
        do_block(1)
        pltpu.sync_copy(
            buf.at[:, pl.ds(0, 1), :],
            out_hbm.at[:, pl.ds(_NBLK * _BR, 1), :])


def kernel(inputs, rpbt):
    del inputs
    tab = jnp.transpose(rpbt).reshape(-1)
    idx = jnp.asarray(_IDX_FLAT)
    return _sc_gather(tab, idx)


# speedup vs baseline: 4.7034x; 1.0697x over previous
import functools

import jax
import jax.numpy as jnp
from jax import lax
from jax.experimental import pallas as pl
from jax.experimental.pallas import tpu as pltpu
from jax.experimental.pallas import tpu_sc as plsc

_NH = 16
_NRD = (2 * 8 - 1) ** 3 + 3
_VP1 = 8 * 8 * 8 + 1
_BR = 8
_NBLK = (_VP1 - 1) // _BR
_NW = 32

_mesh = plsc.VectorSubcoreMesh(core_axis_name="c", subcore_axis_name="s")


@functools.partial(
    pl.kernel,
    mesh=_mesh,
    out_type=jax.ShapeDtypeStruct((_NH, _VP1, _VP1), jnp.float32),
    compiler_params=pltpu.CompilerParams(
        needs_layout_passes=False, use_tc_tiling_on_sc=False),
    scratch_types=[
        pltpu.VMEM((_NH * _NRD,), jnp.float32),
        pltpu.VMEM((_NH, _BR, _VP1), jnp.float32),
    ],
)
def _sc_gather(tab_hbm, out_hbm, tab_v, buf):
    w = lax.axis_index("s") * 2 + lax.axis_index("c")
    pltpu.sync_copy(tab_hbm, tab_v)

    lane = lax.broadcasted_iota(jnp.int32, (16,), 0)
    lane0 = lane < 1
    col_last = jnp.full((16,), _VP1 - 1, jnp.int32)

    def gather_vec(vec):
        return [plsc.load_gather(tab_v, [vec + h * _NRD]) for h in range(_NH)]

    def row_consts(i):
        iv = jnp.broadcast_to(jnp.int32(0) + i, (16,))
        a = iv - 1
        s = (((a >> 6) + 7) * 225 + (((a >> 3) & 7) + 7) * 15
             + ((a & 7) + 7))
        return s, iv == 0

    def rel_idx(s, is_row0, j):
        jc = jnp.minimum(j, _VP1 - 1)
        b = jc - 1
        bsum = (b >> 6) * 225 + ((b >> 3) & 7) * 15 + (b & 7)
        idx = s - bsum
        idx = jnp.where(jc == 0, _NRD - 2, idx)
        return jnp.where(is_row0,
                         jnp.where(jc == 0, _NRD - 1, _NRD - 3),
                         idx)

    def do_block(blk, nrows):
        consts = [row_consts(blk * _BR + r) for r in range(nrows)]

        def body(v, carry):
            j = v * 16 + lane
            for r in range(nrows):
                s, is_row0 = consts[r]
                vals = gather_vec(rel_idx(s, is_row0, j))
                for h in range(_NH):
                    buf[h, r, pl.ds(v * 16, 16)] = vals[h]
            return carry
        lax.fori_loop(0, (_VP1 - 1) // 16, body, 0)

        jt = (_VP1 - 1) + lane
        for r in range(nrows):
            s, is_row0 = consts[r]
            vals = gather_vec(rel_idx(s, is_row0, jt))
            row = jnp.full((16,), r, jnp.int32)
            for h in range(_NH):
                plsc.store_scatter(buf, [jnp.full((16,), h, jnp.int32), row,
                                         col_last], vals[h], mask=lane0)

    for b in range(2):
        blk = w * 2 + b
        do_block(blk, _BR)
        pltpu.sync_copy(buf, out_hbm.at[:, pl.ds(blk * _BR, _BR), :])

    @pl.when(w == _NW - 1)
    def _last_row():
        do_block(_NBLK, 1)
        pltpu.sync_copy(
            buf.at[:, pl.ds(0, 1), :],
            out_hbm.at[:, pl.ds(_NBLK * _BR, 1), :])


def kernel(inputs, rpbt):
    del inputs
    tab = jnp.transpose(rpbt).reshape(-1)
    return _sc_gather(tab)


# speedup vs baseline: 9.9019x; 2.1053x over previous
import functools

import jax
import jax.numpy as jnp
from jax import lax
from jax.experimental import pallas as pl
from jax.experimental.pallas import tpu as pltpu
from jax.experimental.pallas import tpu_sc as plsc

_NH = 16
_HH = 8
_NRD = (2 * 8 - 1) ** 3 + 3
_VP1 = 8 * 8 * 8 + 1
_BR = 8
_NBLK = (_VP1 - 1) // _BR
_NW = 32

_mesh = plsc.VectorSubcoreMesh(core_axis_name="c", subcore_axis_name="s")


@functools.partial(
    pl.kernel,
    mesh=_mesh,
    out_type=jax.ShapeDtypeStruct((_NH, _VP1, _VP1), jnp.float32),
    compiler_params=pltpu.CompilerParams(needs_layout_passes=False),
    scratch_types=[
        pltpu.VMEM((_HH * _NRD,), jnp.float32),
        pltpu.VMEM((_HH, _BR, _VP1), jnp.float32),
    ],
)
def _sc_gather(tab_hbm, out_hbm, tab_v, buf):
    w = lax.axis_index("s") * 2 + lax.axis_index("c")
    h0 = (w & 1) * _HH
    pltpu.sync_copy(tab_hbm.at[pl.ds(h0 * _NRD, _HH * _NRD)], tab_v)

    lane = lax.broadcasted_iota(jnp.int32, (16,), 0)
    lane0 = lane < 1
    col_last = jnp.full((16,), _VP1 - 1, jnp.int32)

    def gather_vec(vec):
        return [plsc.load_gather(tab_v, [vec + h * _NRD]) for h in range(_HH)]

    def row_consts(i):
        iv = jnp.broadcast_to(jnp.int32(0) + i, (16,))
        a = iv - 1
        s = (((a >> 6) + 7) * 225 + (((a >> 3) & 7) + 7) * 15
             + ((a & 7) + 7))
        return s, iv == 0

    def rel_idx(s, is_row0, j):
        jc = jnp.minimum(j, _VP1 - 1)
        b = jc - 1
        bsum = (b >> 6) * 225 + ((b >> 3) & 7) * 15 + (b & 7)
        idx = s - bsum
        idx = jnp.where(jc == 0, _NRD - 2, idx)
        return jnp.where(is_row0,
                         jnp.where(jc == 0, _NRD - 1, _NRD - 3),
                         idx)

    def do_rows(row_of, nrows, r0=0):
        consts = [row_consts(row_of(r)) for r in range(nrows)]

        def body(v, carry):
            j = v * 16 + lane
            for r in range(nrows):
                s, is_row0 = consts[r]
                vals = gather_vec(rel_idx(s, is_row0, j))
                for h in range(_HH):
                    buf[h, r0 + r, pl.ds(v * 16, 16)] = vals[h]
            return carry
        lax.fori_loop(0, (_VP1 - 1) // 16, body, 0)

        jt = (_VP1 - 1) + lane
        for r in range(nrows):
            s, is_row0 = consts[r]
            vals = gather_vec(rel_idx(s, is_row0, jt))
            row = jnp.full((16,), r0 + r, jnp.int32)
            for h in range(_HH):
                plsc.store_scatter(buf, [jnp.full((16,), h, jnp.int32), row,
                                         col_last], vals[h], mask=lane0)

    for k in range(4):
        blk = (w >> 1) * 4 + k
        do_rows(lambda r, blk=blk: blk * _BR + r, _BR)
        pltpu.sync_copy(
            buf, out_hbm.at[pl.ds(h0, _HH), pl.ds(blk * _BR, _BR), :])

    @pl.when(w < 2)
    def _last_row():
        do_rows(lambda r: _NBLK * _BR, 1, r0=_BR - 1)
        pltpu.sync_copy(
            buf.at[:, pl.ds(_BR - 1, 1), :],
            out_hbm.at[pl.ds(h0, _HH), pl.ds(_NBLK * _BR, 1), :])


def kernel(inputs, rpbt):
    del inputs
    tab = jnp.transpose(rpbt).reshape(-1)
    return _sc_gather(tab)


# speedup vs baseline: 14.1638x; 1.4304x over previous
import functools

import jax
import jax.numpy as jnp
from jax import lax
from jax.experimental import pallas as pl
from jax.experimental.pallas import tpu as pltpu
from jax.experimental.pallas import tpu_sc as plsc

_NH = 16
_HH = 8
_NRD = (2 * 8 - 1) ** 3 + 3
_VP1 = 8 * 8 * 8 + 1
_BR = 8
_NBLK = (_VP1 - 1) // _BR
_NW = 32

_mesh = plsc.VectorSubcoreMesh(core_axis_name="c", subcore_axis_name="s")


@functools.partial(
    pl.kernel,
    mesh=_mesh,
    out_type=jax.ShapeDtypeStruct((_VP1, _NH, _VP1), jnp.float32),
    compiler_params=pltpu.CompilerParams(needs_layout_passes=False),
    scratch_types=[
        pltpu.VMEM((_HH * _NRD,), jnp.float32),
        pltpu.VMEM((_BR, _HH, _VP1), jnp.float32),
    ],
)
def _sc_gather(tab_hbm, out_hbm, tab_v, buf):
    w = lax.axis_index("s") * 2 + lax.axis_index("c")
    h0 = (w & 1) * _HH
    pltpu.sync_copy(tab_hbm.at[pl.ds(h0 * _NRD, _HH * _NRD)], tab_v)

    lane = lax.broadcasted_iota(jnp.int32, (16,), 0)
    lane0 = lane < 1
    col_last = jnp.full((16,), _VP1 - 1, jnp.int32)

    def gather_vec(vec):
        return [plsc.load_gather(tab_v, [vec + h * _NRD]) for h in range(_HH)]

    def row_consts(i):
        iv = jnp.broadcast_to(jnp.int32(0) + i, (16,))
        a = iv - 1
        s = (((a >> 6) + 7) * 225 + (((a >> 3) & 7) + 7) * 15
             + ((a & 7) + 7))
        return s, iv == 0

    def rel_idx(s, is_row0, j):
        jc = jnp.minimum(j, _VP1 - 1)
        b = jc - 1
        bsum = (b >> 6) * 225 + ((b >> 3) & 7) * 15 + (b & 7)
        idx = s - bsum
        idx = jnp.where(jc == 0, _NRD - 2, idx)
        return jnp.where(is_row0,
                         jnp.where(jc == 0, _NRD - 1, _NRD - 3),
                         idx)

    def do_rows(row_of, nrows, r0=0):
        consts = [row_consts(row_of(r)) for r in range(nrows)]

        def body(v, carry):
            j = v * 16 + lane
            for r in range(nrows):
                s, is_row0 = consts[r]
                vals = gather_vec(rel_idx(s, is_row0, j))
                for h in range(_HH):
                    buf[r0 + r, h, pl.ds(v * 16, 16)] = vals[h]
            return carry
        lax.fori_loop(0, (_VP1 - 1) // 16, body, 0)

        jt = (_VP1 - 1) + lane
        for r in range(nrows):
            s, is_row0 = consts[r]
            vals = gather_vec(rel_idx(s, is_row0, jt))
            row = jnp.full((16,), r0 + r, jnp.int32)
            for h in range(_HH):
                plsc.store_scatter(buf, [row, jnp.full((16,), h, jnp.int32),
                                         col_last], vals[h], mask=lane0)

    for k in range(4):
        blk = (w >> 1) * 4 + k
        do_rows(lambda r, blk=blk: blk * _BR + r, _BR)
        pltpu.sync_copy(
            buf, out_hbm.at[pl.ds(blk * _BR, _BR), pl.ds(h0, _HH), :])

    @pl.when(w < 2)
    def _last_row():
        do_rows(lambda r: _NBLK * _BR, 1)
        pltpu.sync_copy(
            buf.at[pl.ds(0, 1), :, :],
            out_hbm.at[pl.ds(_NBLK * _BR, 1), pl.ds(h0, _HH), :])


def kernel(inputs, rpbt):
    del inputs
    tab = jnp.transpose(rpbt).reshape(-1)
    out = _sc_gather(tab)
    return jnp.transpose(out, (1, 0, 2))
